# baseline (speedup 1.0000x reference)
import jax
import jax.numpy as jnp
from jax import lax
from jax.experimental import pallas as pl
from jax.experimental.pallas import tpu as pltpu

NUM_EMB = 1024
EMB_DIM = 64
GROUPS = 16
TOTAL_DIM = GROUPS * EMB_DIM
BN = 256


def _sumsq_ref_order(t):
    t = t * t
    a = t[:, 0:8]
    for j in range(1, 8):
        a = a + t[:, 8 * j:8 * j + 8]
    a = a[:, 0:4] + a[:, 4:8]
    a = a[:, 0:2] + a[:, 2:4]
    return a[:, 0:1] + a[:, 1:2]


def _vq_kernel(in_ref, w_ref, wsq_ref, out_ref, idx_ref, loss_ref):
    pid = pl.program_id(0)
    nblocks = pl.num_programs(0)

    loss_part = jnp.float32(0.0)
    for g in range(GROUPS):
        v = in_ref[:, g * EMB_DIM:(g + 1) * EMB_DIM]
        norm = jnp.sqrt(_sumsq_ref_order(v))
        xg = v / (norm + 1e-6)
        x_sq = _sumsq_ref_order(xg)
        wg = w_ref[g]
        scores = lax.dot_general(
            xg, wg, (((1,), (1,)), ((), ())),
            preferred_element_type=jnp.float32)
        dist = (x_sq + wsq_ref[g:g + 1, :]) - 2.0 * scores
        minv = jnp.min(dist, axis=1, keepdims=True)
        iota = lax.broadcasted_iota(jnp.int32, (BN, NUM_EMB), 1)
        idx_col = jnp.min(jnp.where(dist == minv, iota, NUM_EMB),
                          axis=1, keepdims=True)
        idx_ref[:, g:g + 1] = idx_col
        onehot = (iota == idx_col).astype(jnp.float32)
        q = lax.dot_general(
            onehot, wg, (((1,), (0,)), ((), ())),
            preferred_element_type=jnp.float32)
        d_qx = q - xg
        out_ref[:, g * EMB_DIM:(g + 1) * EMB_DIM] = xg + d_qx
        loss_part = loss_part + jnp.sum(d_qx * d_qx)

    prev = jnp.where(pid == 0, jnp.zeros((1, 1), jnp.float32), loss_ref[:, :])
    acc = prev + loss_part
    scale = 1.25 / (GROUPS * 9216 * EMB_DIM)
    loss_ref[:, :] = jnp.where(pid == nblocks - 1, acc * scale, acc)


def kernel(inputs, embed_weights):
    input_shape = inputs.shape
    flat = inputs.reshape(-1, TOTAL_DIM)
    n = flat.shape[0]
    nblocks = n // BN

    w_sq = jnp.sum(embed_weights**2, axis=2)

    out, idx, loss = pl.pallas_call(
        _vq_kernel,
        grid=(nblocks,),
        in_specs=[
            pl.BlockSpec((BN, TOTAL_DIM), lambda i: (i, 0)),
            pl.BlockSpec((GROUPS, NUM_EMB, EMB_DIM), lambda i: (0, 0, 0)),
            pl.BlockSpec((GROUPS, NUM_EMB), lambda i: (0, 0)),
        ],
        out_specs=[
            pl.BlockSpec((BN, TOTAL_DIM), lambda i: (i, 0)),
            pl.BlockSpec((BN, GROUPS), lambda i: (i, 0)),
            pl.BlockSpec((1, 1), lambda i: (0, 0)),
        ],
        out_shape=[
            jax.ShapeDtypeStruct((n, TOTAL_DIM), jnp.float32),
            jax.ShapeDtypeStruct((n, GROUPS), jnp.int32),
            jax.ShapeDtypeStruct((1, 1), jnp.float32),
        ],
    )(flat, embed_weights, w_sq)

    quantized_out = out.reshape(input_shape)
    indices_out = idx.reshape(*input_shape[:-1], GROUPS)
    total_loss = loss[0, 0]
    return (quantized_out, total_loss, indices_out)


# speedup vs baseline: 1.5751x; 1.5751x over previous
import functools

import jax
import jax.numpy as jnp
from jax import lax
from jax.experimental import pallas as pl
from jax.experimental.pallas import tpu as pltpu
from jax.experimental.pallas import tpu_sc as plsc

NUM_EMB = 1024
EMB_DIM = 64
GROUPS = 16
TOTAL_DIM = GROUPS * EMB_DIM
N_TOK = 9216
BN = 256

SC_WORKERS = 32
CHUNK = 128
ROWS_PER_W = N_TOK * GROUPS // SC_WORKERS
NCHUNKS = ROWS_PER_W // CHUNK
TOK_PER_CHUNK = CHUNK // GROUPS


def _tc_argmin_kernel(x_ref, xsq_ref, w_ref, wsq_ref, idx_ref, gidx_ref):
    for g in range(GROUPS):
        xg = x_ref[:, g * EMB_DIM:(g + 1) * EMB_DIM]
        wg = w_ref[g]
        scores = lax.dot_general(
            xg, wg, (((1,), (1,)), ((), ())),
            preferred_element_type=jnp.float32)
        dist = (xsq_ref[:, g:g + 1] + wsq_ref[g:g + 1, :]) - 2.0 * scores
        minv = jnp.min(dist, axis=1, keepdims=True)
        iota = lax.broadcasted_iota(jnp.int32, (BN, NUM_EMB), 1)
        idx_col = jnp.min(jnp.where(dist == minv, iota, NUM_EMB),
                          axis=1, keepdims=True)
        idx_ref[:, g:g + 1] = idx_col
        gidx_ref[:, g:g + 1] = idx_col + g * NUM_EMB


def _sc_gather_body(x_hbm, gidx_hbm, wflat_hbm, out_hbm, part_hbm,
                    idx_v, rows_v, xv, ov, acc_v, sem):
    nc = 2
    wid = lax.axis_index("s") * nc + lax.axis_index("c")

    def chunk_body(c, acc):
        base = wid * ROWS_PER_W + c * CHUNK
        tok = wid * (ROWS_PER_W // GROUPS) + c * TOK_PER_CHUNK
        pltpu.sync_copy(gidx_hbm.at[pl.ds(base, CHUNK)], idx_v)
        pltpu.async_copy(wflat_hbm.at[idx_v], rows_v, sem).wait()
        pltpu.sync_copy(x_hbm.at[pl.ds(tok, TOK_PER_CHUNK), :], xv)

        def row_body(r, acc_in):
            a = acc_in
            xr = r // GROUPS
            xc = (r % GROUPS) * EMB_DIM
            for j in range(EMB_DIM // 16):
                q = rows_v[r, pl.ds(j * 16, 16)]
                xx = xv[xr, pl.ds(xc + j * 16, 16)]
                d = q - xx
                ov[xr, pl.ds(xc + j * 16, 16)] = xx + d
                a = a + d * d
            return a

        acc = lax.fori_loop(0, CHUNK, row_body, acc)
        pltpu.sync_copy(ov, out_hbm.at[pl.ds(tok, TOK_PER_CHUNK), :])
        return acc

    acc = lax.fori_loop(0, NCHUNKS, chunk_body,
                        jnp.zeros((16,), jnp.float32))
    acc_v[...] = acc
    pltpu.sync_copy(acc_v, part_hbm.at[pl.ds(wid * 16, 16)])


def kernel(inputs, embed_weights):
    input_shape = inputs.shape
    flat = inputs.reshape(-1, TOTAL_DIM)
    n = flat.shape[0]
    nblocks = n // BN

    grouped = flat.reshape(-1, GROUPS, EMB_DIM)
    norms = jnp.linalg.norm(grouped, axis=2, keepdims=True)
    den = jnp.repeat((norms + 1e-6).reshape(n, GROUPS), EMB_DIM, axis=1)
    x2d = flat / den
    x_sq = jnp.sum(x2d.reshape(-1, GROUPS, EMB_DIM) ** 2, axis=2)
    w_sq = jnp.sum(embed_weights**2, axis=2)

    idx, gidx = pl.pallas_call(
        _tc_argmin_kernel,
        grid=(nblocks,),
        in_specs=[
            pl.BlockSpec((BN, TOTAL_DIM), lambda i: (i, 0)),
            pl.BlockSpec((BN, GROUPS), lambda i: (i, 0)),
            pl.BlockSpec((GROUPS, NUM_EMB, EMB_DIM), lambda i: (0, 0, 0)),
            pl.BlockSpec((GROUPS, NUM_EMB), lambda i: (0, 0)),
        ],
        out_specs=[
            pl.BlockSpec((BN, GROUPS), lambda i: (i, 0)),
            pl.BlockSpec((BN, GROUPS), lambda i: (i, 0)),
        ],
        out_shape=[
            jax.ShapeDtypeStruct((n, GROUPS), jnp.int32),
            jax.ShapeDtypeStruct((n, GROUPS), jnp.int32),
        ],
    )(x2d, x_sq, embed_weights, w_sq)

    gidx_flat = gidx.reshape(n * GROUPS)
    w_flat = embed_weights.reshape(GROUPS * NUM_EMB, EMB_DIM)
    w_pad = jnp.pad(w_flat, ((0, 0), (0, 128 - EMB_DIM)))

    mesh = plsc.VectorSubcoreMesh(core_axis_name="c", subcore_axis_name="s")
    sc_gather = functools.partial(
        pl.kernel,
        mesh=mesh,
        out_type=[
            jax.ShapeDtypeStruct((n, TOTAL_DIM), jnp.float32),
            jax.ShapeDtypeStruct((SC_WORKERS * 16,), jnp.float32),
        ],
        scratch_types=[
            pltpu.VMEM((CHUNK,), jnp.int32),
            pltpu.VMEM((CHUNK, 128), jnp.float32),
            pltpu.VMEM((TOK_PER_CHUNK, TOTAL_DIM), jnp.float32),
            pltpu.VMEM((TOK_PER_CHUNK, TOTAL_DIM), jnp.float32),
            pltpu.VMEM((16,), jnp.float32),
            pltpu.SemaphoreType.DMA,
        ],
    )(_sc_gather_body)
    out, partials = sc_gather(x2d, gidx_flat, w_pad)

    quantized_out = out.reshape(input_shape)
    indices_out = idx.reshape(*input_shape[:-1], GROUPS)
    scale = 1.25 / (GROUPS * N_TOK * EMB_DIM)
    total_loss = jnp.sum(partials) * scale
    return (quantized_out, total_loss, indices_out)


# speedup vs baseline: 1.7371x; 1.1029x over previous
import functools

import jax
import jax.numpy as jnp
from jax import lax
from jax.experimental import pallas as pl
from jax.experimental.pallas import tpu as pltpu
from jax.experimental.pallas import tpu_sc as plsc

NUM_EMB = 1024
EMB_DIM = 64
GROUPS = 16
TOTAL_DIM = GROUPS * EMB_DIM
N_TOK = 9216
BN = 256

SC_WORKERS = 32
CHUNK = 128
ROWS_PER_W = N_TOK * GROUPS // SC_WORKERS
NCHUNKS = ROWS_PER_W // CHUNK
TOK_PER_CHUNK = CHUNK // GROUPS


def _tc_argmin_kernel(x_ref, xsq_ref, w_ref, wsq_ref, idx_ref, gidx_ref,
                      loss_ref):
    pid = pl.program_id(0)
    nblocks = pl.num_programs(0)
    loss_part = jnp.float32(0.0)
    for g in range(GROUPS):
        xg = x_ref[:, g * EMB_DIM:(g + 1) * EMB_DIM]
        wg = w_ref[g]
        scores = lax.dot_general(
            xg, wg, (((1,), (1,)), ((), ())),
            preferred_element_type=jnp.float32)
        dist = (xsq_ref[:, g:g + 1] + wsq_ref[g:g + 1, :]) - 2.0 * scores
        minv = jnp.min(dist, axis=1, keepdims=True)
        iota = lax.broadcasted_iota(jnp.int32, (BN, NUM_EMB), 1)
        idx_col = jnp.min(jnp.where(dist == minv, iota, NUM_EMB),
                          axis=1, keepdims=True)
        idx_ref[:, g:g + 1] = idx_col
        gidx_ref[:, g:g + 1] = idx_col + g * NUM_EMB
        loss_part = loss_part + jnp.sum(minv)

    prev = jnp.where(pid == 0, jnp.zeros((1, 1), jnp.float32), loss_ref[:, :])
    loss_ref[:, :] = prev + loss_part


def _sc_gather_body(gidx_hbm, wflat_hbm, out_hbm, idx_v, rows_v, ov, sem):
    nc = 2
    wid = lax.axis_index("s") * nc + lax.axis_index("c")

    def chunk_body(c, carry):
        base = wid * ROWS_PER_W + c * CHUNK
        tok = wid * (ROWS_PER_W // GROUPS) + c * TOK_PER_CHUNK
        pltpu.sync_copy(gidx_hbm.at[pl.ds(base, CHUNK)], idx_v)
        pltpu.async_copy(wflat_hbm.at[idx_v], rows_v, sem).wait()

        def row_body(r, carry_in):
            xr = r // GROUPS
            xc = (r % GROUPS) * EMB_DIM
            for j in range(EMB_DIM // 16):
                ov[xr, pl.ds(xc + j * 16, 16)] = rows_v[r, pl.ds(j * 16, 16)]
            return carry_in

        carry = lax.fori_loop(0, CHUNK, row_body, carry)
        pltpu.sync_copy(ov, out_hbm.at[pl.ds(tok, TOK_PER_CHUNK), :])
        return carry

    lax.fori_loop(0, NCHUNKS, chunk_body, jnp.int32(0))


def kernel(inputs, embed_weights):
    input_shape = inputs.shape
    flat = inputs.reshape(-1, TOTAL_DIM)
    n = flat.shape[0]
    nblocks = n // BN

    grouped = flat.reshape(-1, GROUPS, EMB_DIM)
    norms = jnp.linalg.norm(grouped, axis=2, keepdims=True)
    den = jnp.repeat((norms + 1e-6).reshape(n, GROUPS), EMB_DIM, axis=1)
    x2d = flat / den
    x_sq = jnp.sum(x2d.reshape(-1, GROUPS, EMB_DIM) ** 2, axis=2)
    w_sq = jnp.sum(embed_weights**2, axis=2)

    idx, gidx, loss = pl.pallas_call(
        _tc_argmin_kernel,
        grid=(nblocks,),
        in_specs=[
            pl.BlockSpec((BN, TOTAL_DIM), lambda i: (i, 0)),
            pl.BlockSpec((BN, GROUPS), lambda i: (i, 0)),
            pl.BlockSpec((GROUPS, NUM_EMB, EMB_DIM), lambda i: (0, 0, 0)),
            pl.BlockSpec((GROUPS, NUM_EMB), lambda i: (0, 0)),
        ],
        out_specs=[
            pl.BlockSpec((BN, GROUPS), lambda i: (i, 0)),
            pl.BlockSpec((BN, GROUPS), lambda i: (i, 0)),
            pl.BlockSpec((1, 1), lambda i: (0, 0)),
        ],
        out_shape=[
            jax.ShapeDtypeStruct((n, GROUPS), jnp.int32),
            jax.ShapeDtypeStruct((n, GROUPS), jnp.int32),
            jax.ShapeDtypeStruct((1, 1), jnp.float32),
        ],
    )(x2d, x_sq, embed_weights, w_sq)

    gidx_flat = gidx.reshape(n * GROUPS)
    w_flat = embed_weights.reshape(GROUPS * NUM_EMB, EMB_DIM)
    w_pad = jnp.pad(w_flat, ((0, 0), (0, 128 - EMB_DIM)))

    mesh = plsc.VectorSubcoreMesh(core_axis_name="c", subcore_axis_name="s")
    sc_gather = functools.partial(
        pl.kernel,
        mesh=mesh,
        out_type=[
            jax.ShapeDtypeStruct((n, TOTAL_DIM), jnp.float32),
        ],
        scratch_types=[
            pltpu.VMEM((CHUNK,), jnp.int32),
            pltpu.VMEM((CHUNK, 128), jnp.float32),
            pltpu.VMEM((TOK_PER_CHUNK, TOTAL_DIM), jnp.float32),
            pltpu.SemaphoreType.DMA,
        ],
    )(_sc_gather_body)
    (out,) = sc_gather(gidx_flat, w_pad)

    quantized_out = out.reshape(input_shape)
    indices_out = idx.reshape(*input_shape[:-1], GROUPS)
    scale = 1.25 / (GROUPS * N_TOK * EMB_DIM)
    total_loss = loss[0, 0] * scale
    return (quantized_out, total_loss, indices_out)


# speedup vs baseline: 1.8802x; 1.0824x over previous
import functools

import jax
import jax.numpy as jnp
from jax import lax
from jax.experimental import pallas as pl
from jax.experimental.pallas import tpu as pltpu
from jax.experimental.pallas import tpu_sc as plsc

NUM_EMB = 1024
EMB_DIM = 64
GROUPS = 16
TOTAL_DIM = GROUPS * EMB_DIM
N_TOK = 9216
BN = 256

SC_WORKERS = 32
CHUNK = 128
ROWS_PER_W = N_TOK * GROUPS // SC_WORKERS
NCHUNKS = ROWS_PER_W // CHUNK
TOK_PER_CHUNK = CHUNK // GROUPS


def _tc_argmin_kernel(x_ref, xsq_ref, w_ref, wsq_ref, idx_ref, gidx_ref,
                      loss_ref):
    pid = pl.program_id(0)
    nblocks = pl.num_programs(0)
    loss_part = jnp.float32(0.0)
    for g in range(GROUPS):
        xg = x_ref[:, g * EMB_DIM:(g + 1) * EMB_DIM]
        wg = w_ref[g]
        scores = lax.dot_general(
            xg, wg, (((1,), (1,)), ((), ())),
            preferred_element_type=jnp.float32)
        dist = (xsq_ref[:, g:g + 1] + wsq_ref[g:g + 1, :]) - 2.0 * scores
        minv = jnp.min(dist, axis=1, keepdims=True)
        iota = lax.broadcasted_iota(jnp.int32, (BN, NUM_EMB), 1)
        idx_col = jnp.min(jnp.where(dist == minv, iota, NUM_EMB),
                          axis=1, keepdims=True)
        idx_ref[:, g:g + 1] = idx_col
        gidx_ref[:, g:g + 1] = idx_col + g * NUM_EMB
        loss_part = loss_part + jnp.sum(minv)

    prev = jnp.where(pid == 0, jnp.zeros((1, 1), jnp.float32), loss_ref[:, :])
    loss_ref[:, :] = prev + loss_part


def _sc_gather_body(gidx_hbm, wflat_hbm, out_hbm,
                    idx_a, idx_b, rows_a, rows_b, ov, sem_a, sem_b):
    nc = 2
    wid = lax.axis_index("s") * nc + lax.axis_index("c")

    def fetch(c, idx_v, rows_v, sem):
        cm = lax.rem(c, NCHUNKS)
        base = wid * ROWS_PER_W + cm * CHUNK
        pltpu.sync_copy(gidx_hbm.at[pl.ds(base, CHUNK)], idx_v)
        return pltpu.async_copy(wflat_hbm.at[idx_v], rows_v, sem)

    def repack_store(c, rows_v):
        tok = wid * (ROWS_PER_W // GROUPS) + c * TOK_PER_CHUNK

        def row_body(xr, carry_in):
            for rr in range(GROUPS):
                xc = rr * EMB_DIM
                r = xr * GROUPS + rr
                for j in range(EMB_DIM // 16):
                    ov[xr, pl.ds(xc + j * 16, 16)] = (
                        rows_v[r, pl.ds(j * 16, 16)])
            return carry_in

        lax.fori_loop(0, TOK_PER_CHUNK, row_body, jnp.int32(0))
        pltpu.sync_copy(ov, out_hbm.at[pl.ds(tok, TOK_PER_CHUNK), :])

    fetch(0, idx_a, rows_a, sem_a).wait()

    def pair_body(i, carry):
        c0 = 2 * i
        cp_b = fetch(c0 + 1, idx_b, rows_b, sem_b)
        repack_store(c0, rows_a)
        cp_b.wait()
        cp_a = fetch(c0 + 2, idx_a, rows_a, sem_a)
        repack_store(c0 + 1, rows_b)
        cp_a.wait()
        return carry

    lax.fori_loop(0, NCHUNKS // 2, pair_body, jnp.int32(0))


def kernel(inputs, embed_weights):
    input_shape = inputs.shape
    flat = inputs.reshape(-1, TOTAL_DIM)
    n = flat.shape[0]
    nblocks = n // BN

    grouped = flat.reshape(-1, GROUPS, EMB_DIM)
    norms = jnp.linalg.norm(grouped, axis=2, keepdims=True)
    den = jnp.repeat((norms + 1e-6).reshape(n, GROUPS), EMB_DIM, axis=1)
    x2d = flat / den
    x_sq = jnp.sum(x2d.reshape(-1, GROUPS, EMB_DIM) ** 2, axis=2)
    w_sq = jnp.sum(embed_weights**2, axis=2)

    idx, gidx, loss = pl.pallas_call(
        _tc_argmin_kernel,
        grid=(nblocks,),
        in_specs=[
            pl.BlockSpec((BN, TOTAL_DIM), lambda i: (i, 0)),
            pl.BlockSpec((BN, GROUPS), lambda i: (i, 0)),
            pl.BlockSpec((GROUPS, NUM_EMB, EMB_DIM), lambda i: (0, 0, 0)),
            pl.BlockSpec((GROUPS, NUM_EMB), lambda i: (0, 0)),
        ],
        out_specs=[
            pl.BlockSpec((BN, GROUPS), lambda i: (i, 0)),
            pl.BlockSpec((BN, GROUPS), lambda i: (i, 0)),
            pl.BlockSpec((1, 1), lambda i: (0, 0)),
        ],
        out_shape=[
            jax.ShapeDtypeStruct((n, GROUPS), jnp.int32),
            jax.ShapeDtypeStruct((n, GROUPS), jnp.int32),
            jax.ShapeDtypeStruct((1, 1), jnp.float32),
        ],
    )(x2d, x_sq, embed_weights, w_sq)

    gidx_flat = gidx.reshape(n * GROUPS)
    w_flat = embed_weights.reshape(GROUPS * NUM_EMB, EMB_DIM)
    w_pad = jnp.pad(w_flat, ((0, 0), (0, 128 - EMB_DIM)))

    mesh = plsc.VectorSubcoreMesh(core_axis_name="c", subcore_axis_name="s")
    sc_gather = functools.partial(
        pl.kernel,
        mesh=mesh,
        out_type=[
            jax.ShapeDtypeStruct((n, TOTAL_DIM), jnp.float32),
        ],
        scratch_types=[
            pltpu.VMEM((CHUNK,), jnp.int32),
            pltpu.VMEM((CHUNK,), jnp.int32),
            pltpu.VMEM((CHUNK, 128), jnp.float32),
            pltpu.VMEM((CHUNK, 128), jnp.float32),
            pltpu.VMEM((TOK_PER_CHUNK, TOTAL_DIM), jnp.float32),
            pltpu.SemaphoreType.DMA,
            pltpu.SemaphoreType.DMA,
        ],
    )(_sc_gather_body)
    (out,) = sc_gather(gidx_flat, w_pad)

    quantized_out = out.reshape(input_shape)
    indices_out = idx.reshape(*input_shape[:-1], GROUPS)
    scale = 1.25 / (GROUPS * N_TOK * EMB_DIM)
    total_loss = loss[0, 0] * scale
    return (quantized_out, total_loss, indices_out)
